# baseline (speedup 1.0000x reference)
import jax
import jax.numpy as jnp
from jax.experimental import pallas as pl
from jax.experimental.pallas import tpu as pltpu

_KS = 4
_EPS_NORM = 1e-12
_EPS_LN = 1e-5
_H = 16
_DK = 64
_MC = 256
_TL = 256

_SEM_PROJ = ("parallel", "parallel")
_SEM_REC = ("parallel",)
_VMEM = 56 * 1024 * 1024


def _dot_x2(a, bmat):
    hi = a.astype(jnp.bfloat16).astype(jnp.float32)
    lo = a - hi
    return (jnp.dot(hi, bmat, preferred_element_type=jnp.float32)
            + jnp.dot(lo, bmat, preferred_element_type=jnp.float32))


def _proj_body(x_ref, xp_ref, w_ref, cw_ref, g_ref, gt_ref, o_ref):
    j = pl.program_id(1)
    x = x_ref[0]
    y = jnp.dot(x, w_ref[...], preferred_element_type=jnp.float32)
    xh = xp_ref[0, _TL - _KS + 1 :, :]
    yh = jnp.dot(xh, w_ref[...], preferred_element_type=jnp.float32)[:, :3072]
    yh = yh * jnp.where(j > 0, 1.0, 0.0)

    z = y[:, :3072]
    acc = z * (1.0 + cw_ref[_KS - 1 : _KS, :])
    for s in range(1, _KS):
        zs = jnp.concatenate([yh[_KS - 1 - s :], z[: _TL - s]], axis=0)
        acc = acc + zs * cw_ref[_KS - 1 - s : _KS - s, :]
    z = acc * jax.nn.sigmoid(acc)

    zqk = z[:, :2048]
    s32 = _dot_x2(zqk * zqk, g_ref[...])
    scale = _dot_x2(jax.lax.rsqrt(s32 + _EPS_NORM), gt_ref[...])
    o_ref[0, :, :2048] = zqk * scale
    o_ref[0, :, 2048:3072] = z[:, 2048:]
    o_ref[0, :, 3072:] = y[:, 3072:]


def _rec_body(q_ref, k_ref, v_ref, w0_ref, o_ref):
    L = q_ref.shape[1]
    n_mc = L // _MC

    q = q_ref[0]
    k = k_ref[0]
    v = v_ref[0]

    lane = jax.lax.broadcasted_iota(jnp.int32, (1, 128), 1)
    left = lane < _DK
    rows = jax.lax.broadcasted_iota(jnp.int32, (_MC, _MC), 0)
    cols = jax.lax.broadcasted_iota(jnp.int32, (_MC, _MC), 1)
    causal = rows >= cols
    r128 = jax.lax.broadcasted_iota(jnp.int32, (128, 128), 0)
    c128 = jax.lax.broadcasted_iota(jnp.int32, (128, 128), 1)
    blockdiag = (r128 // _DK) == (c128 // _DK)

    W = w0_ref[0]
    for i in range(n_mc):
        sl_ = slice(i * _MC, (i + 1) * _MC)
        qq, kk, vv = q[sl_], k[sl_], v[sl_]
        inter = jnp.dot(qq, W, preferred_element_type=jnp.float32)
        aa = jax.lax.dot_general(qq[:, :_DK], kk[:, :_DK],
                                 (((1,), (1,)), ((), ())),
                                 preferred_element_type=jnp.float32)
        ab = jax.lax.dot_general(qq[:, _DK:], kk[:, _DK:],
                                 (((1,), (1,)), ((), ())),
                                 preferred_element_type=jnp.float32)
        aa = jnp.where(causal, aa, 0.0)
        ab = jnp.where(causal, ab, 0.0)
        intra = (jnp.dot(aa, jnp.where(left, vv, 0.0),
                         preferred_element_type=jnp.float32)
                 + jnp.dot(ab, jnp.where(left, 0.0, vv),
                           preferred_element_type=jnp.float32))
        upd = jax.lax.dot_general(kk, vv, (((0,), (0,)), ((), ())),
                                  preferred_element_type=jnp.float32)
        o_ref[0, sl_, :] = inter + intra
        W = W + jnp.where(blockdiag, upd, 0.0)


def _out_body(o_ref, g_ref, g64_ref, g64t_ref, gam_ref, bet_ref, wo_ref,
              out_ref):
    o = o_ref[0]
    inv = 1.0 / _DK
    mu = _dot_x2(_dot_x2(o, g64_ref[...]) * inv, g64t_ref[...])
    d = o - mu
    v16 = _dot_x2(d * d, g64_ref[...]) * inv
    scale = _dot_x2(jax.lax.rsqrt(v16 + _EPS_LN), g64t_ref[...])
    o_ln = d * scale * gam_ref[...] + bet_ref[...]
    out_ref[0] = jnp.dot(o_ln * g_ref[0], wo_ref[...],
                         preferred_element_type=jnp.float32)


def kernel(hidden_states, Wq, Wk, Wv, conv_q, conv_k, conv_v,
           W_init, ln_gamma, ln_beta, Wg, Wo):
    b, l, dim = hidden_states.shape
    h, dk = _H, _DK
    npair = h // 2
    nt = l // _TL

    W4 = jnp.concatenate([Wq, Wk, Wv, Wg], axis=0).T
    cw3 = jnp.concatenate([conv_q, conv_k, conv_v], axis=0).T
    eye32 = jnp.eye(2 * h, dtype=jnp.float32)
    G = jnp.repeat(eye32, dk, axis=0)
    GT = G.T
    eye16 = jnp.eye(h, dtype=jnp.float32)
    G64 = jnp.repeat(eye16, dk, axis=0)
    G64T = G64.T
    gam = jnp.tile(ln_gamma, h).reshape(1, dim)
    bet = jnp.tile(ln_beta, h).reshape(1, dim)

    wp = W_init[0].reshape(npair, 2, dk, dk)
    w0 = jnp.zeros((npair, 2, dk, 2, dk), jnp.float32)
    w0 = w0.at[:, 0, :, 0, :].set(wp[:, 0]).at[:, 1, :, 1, :].set(wp[:, 1])
    w0 = w0.reshape(npair, 2 * dk, 2 * dk)

    y4 = pl.pallas_call(
        _proj_body,
        grid=(b, nt),
        in_specs=[
            pl.BlockSpec((1, _TL, dim), lambda i, j: (i, j, 0)),
            pl.BlockSpec((1, _TL, dim),
                         lambda i, j: (i, jnp.maximum(j - 1, 0), 0)),
            pl.BlockSpec((dim, 4 * dim), lambda i, j: (0, 0)),
            pl.BlockSpec((_KS, 3 * dim), lambda i, j: (0, 0)),
            pl.BlockSpec((2 * dim, 2 * h), lambda i, j: (0, 0)),
            pl.BlockSpec((2 * h, 2 * dim), lambda i, j: (0, 0)),
        ],
        out_specs=pl.BlockSpec((1, _TL, 4 * dim), lambda i, j: (i, j, 0)),
        out_shape=jax.ShapeDtypeStruct((b, l, 4 * dim), jnp.float32),
        compiler_params=pltpu.CompilerParams(
            dimension_semantics=_SEM_PROJ,
            vmem_limit_bytes=_VMEM,
        ),
        name="titans_qkvg_proj",
    )(hidden_states, hidden_states, W4, cw3, G, GT)

    o_pre = pl.pallas_call(
        _rec_body,
        grid=(b * npair,),
        in_specs=[
            pl.BlockSpec((1, l, 2 * dk), lambda c: (c // npair, 0, c % npair)),
            pl.BlockSpec((1, l, 2 * dk),
                         lambda c: (c // npair, 0, npair + c % npair)),
            pl.BlockSpec((1, l, 2 * dk),
                         lambda c: (c // npair, 0, 2 * npair + c % npair)),
            pl.BlockSpec((1, 2 * dk, 2 * dk), lambda c: (c % npair, 0, 0)),
        ],
        out_specs=pl.BlockSpec((1, l, 2 * dk),
                               lambda c: (c // npair, 0, c % npair)),
        out_shape=jax.ShapeDtypeStruct((b, l, dim), jnp.float32),
        compiler_params=pltpu.CompilerParams(
            dimension_semantics=_SEM_REC,
            vmem_limit_bytes=_VMEM,
        ),
        name="titans_recurrence",
    )(y4, y4, y4, w0)

    out = pl.pallas_call(
        _out_body,
        grid=(b, nt),
        in_specs=[
            pl.BlockSpec((1, _TL, dim), lambda i, j: (i, j, 0)),
            pl.BlockSpec((1, _TL, dim), lambda i, j: (i, j, 3)),
            pl.BlockSpec((dim, h), lambda i, j: (0, 0)),
            pl.BlockSpec((h, dim), lambda i, j: (0, 0)),
            pl.BlockSpec((1, dim), lambda i, j: (0, 0)),
            pl.BlockSpec((1, dim), lambda i, j: (0, 0)),
            pl.BlockSpec((dim, dim), lambda i, j: (0, 0)),
        ],
        out_specs=pl.BlockSpec((1, _TL, dim), lambda i, j: (i, j, 0)),
        out_shape=jax.ShapeDtypeStruct((b, l, dim), jnp.float32),
        compiler_params=pltpu.CompilerParams(
            dimension_semantics=_SEM_PROJ,
            vmem_limit_bytes=_VMEM,
        ),
        name="titans_out_proj",
    )(o_pre, y4, G64, G64T, gam, bet, Wo.T)
    return out


# speedup vs baseline: 1.0534x; 1.0534x over previous
import jax
import jax.numpy as jnp
from jax.experimental import pallas as pl
from jax.experimental.pallas import tpu as pltpu

_KS = 4
_EPS_NORM = 1e-12
_EPS_LN = 1e-5
_H = 16
_DK = 64
_MC = 256
_TL = 256

_SEM_PROJ = ("parallel", "parallel")
_SEM_REC = ("parallel",)
_VMEM = 56 * 1024 * 1024


def _dot_x2(a, bmat):
    hi = a.astype(jnp.bfloat16).astype(jnp.float32)
    lo = a - hi
    return (jnp.dot(hi, bmat, preferred_element_type=jnp.float32)
            + jnp.dot(lo, bmat, preferred_element_type=jnp.float32))


def _proj_body(x_ref, xp_ref, w_ref, cw_ref, g_ref, gt_ref, o_ref):
    j = pl.program_id(1)
    x = x_ref[0]
    y = jnp.dot(x, w_ref[...], preferred_element_type=jnp.float32)
    xh = xp_ref[0, _TL - _KS + 1 :, :]
    yh = jnp.dot(xh, w_ref[...], preferred_element_type=jnp.float32)[:, :3072]
    yh = yh * jnp.where(j > 0, 1.0, 0.0)

    z = y[:, :3072]
    acc = z * (1.0 + cw_ref[_KS - 1 : _KS, :])
    for s in range(1, _KS):
        zs = jnp.concatenate([yh[_KS - 1 - s :], z[: _TL - s]], axis=0)
        acc = acc + zs * cw_ref[_KS - 1 - s : _KS - s, :]
    z = acc * jax.nn.sigmoid(acc)

    zqk = z[:, :2048]
    s32 = jnp.dot(zqk * zqk, g_ref[...], preferred_element_type=jnp.float32)
    scale = _dot_x2(jax.lax.rsqrt(s32 + _EPS_NORM), gt_ref[...])
    o_ref[0, :, :2048] = zqk * scale
    o_ref[0, :, 2048:3072] = z[:, 2048:]
    o_ref[0, :, 3072:] = y[:, 3072:]


def _rec_body(q_ref, k_ref, v_ref, w0_ref, o_ref):
    L = q_ref.shape[1]
    n_mc = L // _MC

    q = q_ref[0]
    k = k_ref[0]
    v = v_ref[0]

    lane = jax.lax.broadcasted_iota(jnp.int32, (1, 128), 1)
    left = lane < _DK
    rows = jax.lax.broadcasted_iota(jnp.int32, (_MC, _MC), 0)
    cols = jax.lax.broadcasted_iota(jnp.int32, (_MC, _MC), 1)
    causal = rows >= cols
    r128 = jax.lax.broadcasted_iota(jnp.int32, (128, 128), 0)
    c128 = jax.lax.broadcasted_iota(jnp.int32, (128, 128), 1)
    blockdiag = (r128 // _DK) == (c128 // _DK)

    W = w0_ref[0]
    for i in range(n_mc):
        sl_ = slice(i * _MC, (i + 1) * _MC)
        qq, kk, vv = q[sl_], k[sl_], v[sl_]
        inter = jnp.dot(qq, W, preferred_element_type=jnp.float32)
        aa = jax.lax.dot_general(qq[:, :_DK], kk[:, :_DK],
                                 (((1,), (1,)), ((), ())),
                                 preferred_element_type=jnp.float32)
        ab = jax.lax.dot_general(qq[:, _DK:], kk[:, _DK:],
                                 (((1,), (1,)), ((), ())),
                                 preferred_element_type=jnp.float32)
        aa = jnp.where(causal, aa, 0.0)
        ab = jnp.where(causal, ab, 0.0)
        intra = (jnp.dot(aa, jnp.where(left, vv, 0.0),
                         preferred_element_type=jnp.float32)
                 + jnp.dot(ab, jnp.where(left, 0.0, vv),
                           preferred_element_type=jnp.float32))
        upd = jax.lax.dot_general(kk, vv, (((0,), (0,)), ((), ())),
                                  preferred_element_type=jnp.float32)
        o_ref[0, sl_, :] = inter + intra
        W = W + jnp.where(blockdiag, upd, 0.0)


def _out_body(o_ref, g_ref, g64_ref, g64t_ref, gam_ref, bet_ref, wo_ref,
              out_ref):
    o = o_ref[0]
    inv = 1.0 / _DK
    mu = _dot_x2(jnp.dot(o, g64_ref[...], preferred_element_type=jnp.float32)
                 * inv, g64t_ref[...])
    d = o - mu
    v16 = jnp.dot(d * d, g64_ref[...], preferred_element_type=jnp.float32) * inv
    scale = _dot_x2(jax.lax.rsqrt(v16 + _EPS_LN), g64t_ref[...])
    o_ln = d * scale * gam_ref[...] + bet_ref[...]
    out_ref[0] = jnp.dot(o_ln * g_ref[0], wo_ref[...],
                         preferred_element_type=jnp.float32)


def kernel(hidden_states, Wq, Wk, Wv, conv_q, conv_k, conv_v,
           W_init, ln_gamma, ln_beta, Wg, Wo):
    b, l, dim = hidden_states.shape
    h, dk = _H, _DK
    npair = h // 2
    nt = l // _TL

    W4 = jnp.concatenate([Wq, Wk, Wv, Wg], axis=0).T
    cw3 = jnp.concatenate([conv_q, conv_k, conv_v], axis=0).T
    eye32 = jnp.eye(2 * h, dtype=jnp.float32)
    G = jnp.repeat(eye32, dk, axis=0)
    GT = G.T
    eye16 = jnp.eye(h, dtype=jnp.float32)
    G64 = jnp.repeat(eye16, dk, axis=0)
    G64T = G64.T
    gam = jnp.tile(ln_gamma, h).reshape(1, dim)
    bet = jnp.tile(ln_beta, h).reshape(1, dim)

    wp = W_init[0].reshape(npair, 2, dk, dk)
    w0 = jnp.zeros((npair, 2, dk, 2, dk), jnp.float32)
    w0 = w0.at[:, 0, :, 0, :].set(wp[:, 0]).at[:, 1, :, 1, :].set(wp[:, 1])
    w0 = w0.reshape(npair, 2 * dk, 2 * dk)

    y4 = pl.pallas_call(
        _proj_body,
        grid=(b, nt),
        in_specs=[
            pl.BlockSpec((1, _TL, dim), lambda i, j: (i, j, 0)),
            pl.BlockSpec((1, _TL, dim),
                         lambda i, j: (i, jnp.maximum(j - 1, 0), 0)),
            pl.BlockSpec((dim, 4 * dim), lambda i, j: (0, 0)),
            pl.BlockSpec((_KS, 3 * dim), lambda i, j: (0, 0)),
            pl.BlockSpec((2 * dim, 2 * h), lambda i, j: (0, 0)),
            pl.BlockSpec((2 * h, 2 * dim), lambda i, j: (0, 0)),
        ],
        out_specs=pl.BlockSpec((1, _TL, 4 * dim), lambda i, j: (i, j, 0)),
        out_shape=jax.ShapeDtypeStruct((b, l, 4 * dim), jnp.float32),
        compiler_params=pltpu.CompilerParams(
            dimension_semantics=_SEM_PROJ,
            vmem_limit_bytes=_VMEM,
        ),
        name="titans_qkvg_proj",
    )(hidden_states, hidden_states, W4, cw3, G, GT)

    o_pre = pl.pallas_call(
        _rec_body,
        grid=(b * npair,),
        in_specs=[
            pl.BlockSpec((1, l, 2 * dk), lambda c: (c // npair, 0, c % npair)),
            pl.BlockSpec((1, l, 2 * dk),
                         lambda c: (c // npair, 0, npair + c % npair)),
            pl.BlockSpec((1, l, 2 * dk),
                         lambda c: (c // npair, 0, 2 * npair + c % npair)),
            pl.BlockSpec((1, 2 * dk, 2 * dk), lambda c: (c % npair, 0, 0)),
        ],
        out_specs=pl.BlockSpec((1, l, 2 * dk),
                               lambda c: (c // npair, 0, c % npair)),
        out_shape=jax.ShapeDtypeStruct((b, l, dim), jnp.float32),
        compiler_params=pltpu.CompilerParams(
            dimension_semantics=_SEM_REC,
            vmem_limit_bytes=_VMEM,
        ),
        name="titans_recurrence",
    )(y4, y4, y4, w0)

    out = pl.pallas_call(
        _out_body,
        grid=(b, nt),
        in_specs=[
            pl.BlockSpec((1, _TL, dim), lambda i, j: (i, j, 0)),
            pl.BlockSpec((1, _TL, dim), lambda i, j: (i, j, 3)),
            pl.BlockSpec((dim, h), lambda i, j: (0, 0)),
            pl.BlockSpec((h, dim), lambda i, j: (0, 0)),
            pl.BlockSpec((1, dim), lambda i, j: (0, 0)),
            pl.BlockSpec((1, dim), lambda i, j: (0, 0)),
            pl.BlockSpec((dim, dim), lambda i, j: (0, 0)),
        ],
        out_specs=pl.BlockSpec((1, _TL, dim), lambda i, j: (i, j, 0)),
        out_shape=jax.ShapeDtypeStruct((b, l, dim), jnp.float32),
        compiler_params=pltpu.CompilerParams(
            dimension_semantics=_SEM_PROJ,
            vmem_limit_bytes=_VMEM,
        ),
        name="titans_out_proj",
    )(o_pre, y4, G64, G64T, gam, bet, Wo.T)
    return out
